# baseline (speedup 1.0000x reference)
import functools

import jax
import jax.numpy as jnp
from jax import lax
from jax.experimental import pallas as pl
from jax.experimental.pallas import tpu as pltpu
from jax.experimental.pallas import tpu_sc as plsc

N = 10000
E = 320000
D = 128

NC = 2
NS = 16
NW = NC * NS
RPW = 312
RPW_LAST = N - (NW - 1) * RPW
RPW_MAX = max(RPW, RPW_LAST)
CHUNK = 6400
GB = 64
SBG = 16
SBE = SBG * GB
EPAD = 313 * SBE


_ROWS = 400


def _dense1_body(x_ref, wemb_ref, bemb_ref, wp_ref, bp_ref, h_ref, m_ref):
    h = jnp.dot(x_ref[:], wemb_ref[:], preferred_element_type=jnp.float32)
    h = h + bemb_ref[:]
    h_ref[:] = h
    m = jnp.dot(h, wp_ref[:], preferred_element_type=jnp.float32) + bp_ref[:]
    m_ref[:] = jnp.maximum(m, 0.0)


def _dense2_body(h_ref, agg_ref, ws_ref, wn_ref, b_ref, wp_ref, bp_ref,
                 h2_ref, m2_ref):
    h2 = (jnp.dot(h_ref[:], ws_ref[:], preferred_element_type=jnp.float32)
          + jnp.dot(agg_ref[:], wn_ref[:], preferred_element_type=jnp.float32)
          + b_ref[:])
    h2_ref[:] = h2
    m2 = jnp.dot(h2, wp_ref[:], preferred_element_type=jnp.float32) + bp_ref[:]
    m2_ref[:] = jnp.maximum(m2, 0.0)


def _dense3_body(h_ref, agg_ref, ws_ref, wn_ref, b_ref, out_ref):
    out_ref[:] = (jnp.dot(h_ref[:], ws_ref[:], preferred_element_type=jnp.float32)
                  + jnp.dot(agg_ref[:], wn_ref[:], preferred_element_type=jnp.float32)
                  + b_ref[:])


def _row_spec():
    return pl.BlockSpec((_ROWS, D), lambda i: (i, 0))


def _w_spec():
    return pl.BlockSpec((D, D), lambda i: (0, 0))


def _b_spec():
    return pl.BlockSpec((1, D), lambda i: (0, 0))


def _dense1(x, W_emb, b_emb, W_pool, b_pool):
    return pl.pallas_call(
        _dense1_body,
        grid=(N // _ROWS,),
        in_specs=[_row_spec(), _w_spec(), _b_spec(), _w_spec(), _b_spec()],
        out_specs=[_row_spec(), _row_spec()],
        out_shape=[jax.ShapeDtypeStruct((N, D), jnp.float32),
                   jax.ShapeDtypeStruct((N, D), jnp.float32)],
    )(x, W_emb, b_emb.reshape(1, D), W_pool, b_pool.reshape(1, D))


def _dense2(h, agg, W_self, W_neigh, bias, W_pool, b_pool):
    return pl.pallas_call(
        _dense2_body,
        grid=(N // _ROWS,),
        in_specs=[_row_spec(), _row_spec(), _w_spec(), _w_spec(), _b_spec(),
                  _w_spec(), _b_spec()],
        out_specs=[_row_spec(), _row_spec()],
        out_shape=[jax.ShapeDtypeStruct((N, D), jnp.float32),
                   jax.ShapeDtypeStruct((N, D), jnp.float32)],
    )(h, agg, W_self, W_neigh, bias.reshape(1, D), W_pool, b_pool.reshape(1, D))


def _dense3(h, agg, W_self, W_neigh, bias):
    return pl.pallas_call(
        _dense3_body,
        grid=(N // _ROWS,),
        in_specs=[_row_spec(), _row_spec(), _w_spec(), _w_spec(), _b_spec()],
        out_specs=_row_spec(),
        out_shape=jax.ShapeDtypeStruct((N, D), jnp.float32),
    )(h, agg, W_self, W_neigh, bias.reshape(1, D))


_SROWS = 624


def _segmax_body(edge_ref, m_ref, out_ref, src_v, dst_v, csrc, cdst,
                 rows_v, acc, sem):
    cid = lax.axis_index("c")
    sid = lax.axis_index("s")
    wid = sid * NC + cid
    lo = wid * RPW
    hi = jnp.where(wid == NW - 1, N, lo + RPW)
    lane = jnp.arange(16, dtype=jnp.int32)
    zeros = jnp.zeros((16,), jnp.float32)
    pad_src = wid * 16 + lane

    def _zero(r, _):
        for cb in range(D // 16):
            acc[r, pl.ds(cb * 16, 16)] = zeros
        return 0
    lax.fori_loop(0, RPW_MAX + 1, _zero, 0)

    pad_dst = jnp.full((16,), RPW_MAX, jnp.int32)

    def _chunk(ch, _):
        off = ch * CHUNK
        pltpu.sync_copy(edge_ref.at[0, pl.ds(off, CHUNK)], src_v)
        pltpu.sync_copy(edge_ref.at[1, pl.ds(off, CHUNK)], dst_v)

        def _scan(i, cntv):
            dstv = dst_v[pl.ds(i * 16, 16)]
            srcv = src_v[pl.ds(i * 16, 16)]
            msk = (dstv >= lo) & (dstv < hi)
            cs = plsc.cumsum(jnp.where(msk, 1, 0))
            pos = cs - 1 + cntv
            plsc.store_scatter(csrc, [pos], srcv, mask=msk)
            plsc.store_scatter(cdst, [pos], dstv - lo, mask=msk)
            pc = plsc.all_reduce_population_count(msk)
            return cntv + pc

        cntv = lax.fori_loop(0, CHUNK // 16, _scan,
                             jnp.zeros((16,), jnp.int32))
        cnt = lax.reduce_max(cntv, axes=(0,))
        for k in range(GB // 16):
            ppos = cnt + k * 16 + lane
            plsc.store_scatter(csrc, [ppos], pad_src)
            plsc.store_scatter(cdst, [ppos], pad_dst)
        ngrp = (cnt + GB - 1) // GB

        def _fire(j, slot):
            idx = csrc.at[pl.ds(j * GB, GB)]
            return pltpu.make_async_copy(m_ref.at[idx], rows_v.at[slot],
                                         sem.at[slot])

        @pl.when(ngrp > 0)
        def _():
            _fire(0, 0).start()

        def _grp(j, _):
            slot = lax.rem(j, 2)

            @pl.when(j + 1 < ngrp)
            def _():
                _fire(j + 1, 1 - slot).start()

            _fire(j, slot).wait()
            for sub in range(GB // 16):
                dstv = cdst[pl.ds(j * GB + sub * 16, 16)]
                dstl = [dstv[e] for e in range(16)]
                for cb in range(D // 16):
                    for e in range(16):
                        r = sub * 16 + e
                        cur = acc[dstl[e], pl.ds(cb * 16, 16)]
                        val = rows_v[slot, r, pl.ds(cb * 16, 16)]
                        acc[dstl[e], pl.ds(cb * 16, 16)] = jnp.maximum(cur, val)
            return 0

        lax.fori_loop(0, ngrp, _grp, 0)
        return 0

    lax.fori_loop(0, E // CHUNK, _chunk, 0)

    @pl.when(wid < NW - 1)
    def _():
        pltpu.sync_copy(acc.at[pl.ds(0, RPW)], out_ref.at[pl.ds(lo, RPW)])

    @pl.when(wid == NW - 1)
    def _():
        pltpu.sync_copy(acc.at[pl.ds(0, RPW_LAST)],
                        out_ref.at[pl.ds((NW - 1) * RPW, RPW_LAST)])


@functools.partial(
    pl.kernel,
    out_type=jax.ShapeDtypeStruct((N, D), jnp.float32),
    mesh=plsc.VectorSubcoreMesh(core_axis_name="c", subcore_axis_name="s",
                                num_cores=NC, num_subcores=NS),
    compiler_params=pltpu.CompilerParams(needs_layout_passes=False),
    scratch_types=[
        pltpu.VMEM((CHUNK,), jnp.int32),
        pltpu.VMEM((CHUNK,), jnp.int32),
        pltpu.VMEM((CHUNK + GB,), jnp.int32),
        pltpu.VMEM((CHUNK + GB,), jnp.int32),
        pltpu.VMEM((2, GB, D), jnp.float32),
        pltpu.VMEM((RPW_MAX + 1, D), jnp.float32),
        pltpu.SemaphoreType.DMA((2,)),
    ],
)
def _segmax(edge_ref, m_ref, out_ref, src_v, dst_v, csrc, cdst,
            rows_v, acc, sem):
    _segmax_body(edge_ref, m_ref, out_ref, src_v, dst_v, csrc, cdst,
                 rows_v, acc, sem)


def kernel(x, edge_index, W_emb, b_emb, W_pool1, b_pool1, W_self1, W_neigh1,
           bias1, W_pool2, b_pool2, W_self2, W_neigh2, bias2):
    h, m1 = _dense1(x, W_emb, b_emb, W_pool1, b_pool1)
    agg1 = _segmax(edge_index, m1)
    h2, m2 = _dense2(h, agg1, W_self1, W_neigh1, bias1, W_pool2, b_pool2)
    agg2 = _segmax(edge_index, m2)
    return _dense3(h2, agg2, W_self2, W_neigh2, bias2)


# speedup vs baseline: 1.1932x; 1.1932x over previous
import functools

import jax
import jax.numpy as jnp
from jax import lax
from jax.experimental import pallas as pl
from jax.experimental.pallas import tpu as pltpu
from jax.experimental.pallas import tpu_sc as plsc

N = 10000
E = 320000
D = 128

NC = 2
NS = 16
NW = NC * NS
RPW = 312
RPW_LAST = N - (NW - 1) * RPW
RPW_MAX = max(RPW, RPW_LAST)
CHUNK = 6400
GB = 64
CAP = 8192


_ROWS = 400


def _dense1_body(x_ref, wemb_ref, bemb_ref, wp_ref, bp_ref, h_ref, m_ref):
    h = jnp.dot(x_ref[:], wemb_ref[:], preferred_element_type=jnp.float32)
    h = h + bemb_ref[:]
    h_ref[:] = h
    m = jnp.dot(h, wp_ref[:], preferred_element_type=jnp.float32) + bp_ref[:]
    m_ref[:] = jnp.maximum(m, 0.0)


def _dense2_body(h_ref, agg_ref, ws_ref, wn_ref, b_ref, wp_ref, bp_ref,
                 h2_ref, m2_ref):
    h2 = (jnp.dot(h_ref[:], ws_ref[:], preferred_element_type=jnp.float32)
          + jnp.dot(agg_ref[:], wn_ref[:], preferred_element_type=jnp.float32)
          + b_ref[:])
    h2_ref[:] = h2
    m2 = jnp.dot(h2, wp_ref[:], preferred_element_type=jnp.float32) + bp_ref[:]
    m2_ref[:] = jnp.maximum(m2, 0.0)


def _dense3_body(h_ref, agg_ref, ws_ref, wn_ref, b_ref, out_ref):
    out_ref[:] = (jnp.dot(h_ref[:], ws_ref[:], preferred_element_type=jnp.float32)
                  + jnp.dot(agg_ref[:], wn_ref[:], preferred_element_type=jnp.float32)
                  + b_ref[:])


def _row_spec():
    return pl.BlockSpec((_ROWS, D), lambda i: (i, 0))


def _w_spec():
    return pl.BlockSpec((D, D), lambda i: (0, 0))


def _b_spec():
    return pl.BlockSpec((1, D), lambda i: (0, 0))


def _dense1(x, W_emb, b_emb, W_pool, b_pool):
    return pl.pallas_call(
        _dense1_body,
        grid=(N // _ROWS,),
        in_specs=[_row_spec(), _w_spec(), _b_spec(), _w_spec(), _b_spec()],
        out_specs=[_row_spec(), _row_spec()],
        out_shape=[jax.ShapeDtypeStruct((N, D), jnp.float32),
                   jax.ShapeDtypeStruct((N, D), jnp.float32)],
    )(x, W_emb, b_emb.reshape(1, D), W_pool, b_pool.reshape(1, D))


def _dense2(h, agg, W_self, W_neigh, bias, W_pool, b_pool):
    return pl.pallas_call(
        _dense2_body,
        grid=(N // _ROWS,),
        in_specs=[_row_spec(), _row_spec(), _w_spec(), _w_spec(), _b_spec(),
                  _w_spec(), _b_spec()],
        out_specs=[_row_spec(), _row_spec()],
        out_shape=[jax.ShapeDtypeStruct((N, D), jnp.float32),
                   jax.ShapeDtypeStruct((N, D), jnp.float32)],
    )(h, agg, W_self, W_neigh, bias.reshape(1, D), W_pool, b_pool.reshape(1, D))


def _dense3(h, agg, W_self, W_neigh, bias):
    return pl.pallas_call(
        _dense3_body,
        grid=(N // _ROWS,),
        in_specs=[_row_spec(), _row_spec(), _w_spec(), _w_spec(), _b_spec()],
        out_specs=_row_spec(),
        out_shape=jax.ShapeDtypeStruct((N, D), jnp.float32),
    )(h, agg, W_self, W_neigh, bias.reshape(1, D))


def _segmax_body(edge_ref, m_ref, out_ref, src_v, dst_v, csrc, cdst,
                 rows_v, acc, sem):
    cid = lax.axis_index("c")
    sid = lax.axis_index("s")
    wid = sid * NC + cid
    lo = wid * RPW
    hi = jnp.where(wid == NW - 1, N, lo + RPW)
    lane = jnp.arange(16, dtype=jnp.int32)
    zeros = jnp.zeros((16,), jnp.float32)
    pad_src = wid * 16 + lane

    def _zero(r, _):
        for cb in range(D // 16):
            acc[r, pl.ds(cb * 16, 16)] = zeros
        return 0
    lax.fori_loop(0, RPW_MAX + 1, _zero, 0)

    pad_dst = jnp.full((16,), RPW_MAX, jnp.int32)

    def _fire(g, slot):
        start = (g & (CAP // GB - 1)) * GB
        idx = csrc.at[pl.ds(start, GB)]
        return pltpu.make_async_copy(m_ref.at[idx], rows_v.at[slot],
                                     sem.at[slot])

    def _run_groups(g0, n):
        @pl.when(n > 0)
        def _():
            _fire(g0, 0).start()

        def _grp(i, _):
            slot = lax.rem(i, 2)

            @pl.when(i + 1 < n)
            def _():
                _fire(g0 + i + 1, 1 - slot).start()

            _fire(g0 + i, slot).wait()
            start = ((g0 + i) & (CAP // GB - 1)) * GB
            for sub in range(GB // 16):
                dstv = cdst[pl.ds(start + sub * 16, 16)]
                for e in range(16):
                    dstl = dstv[e]
                    r = sub * 16 + e
                    for cb in range(D // 16):
                        cur = acc[dstl, pl.ds(cb * 16, 16)]
                        val = rows_v[slot, r, pl.ds(cb * 16, 16)]
                        acc[dstl, pl.ds(cb * 16, 16)] = jnp.maximum(cur, val)
            return 0

        lax.fori_loop(0, n, _grp, 0)

    def _chunk(ch, carry):
        cntv, gdonev = carry
        off = ch * CHUNK
        pltpu.sync_copy(edge_ref.at[0, pl.ds(off, CHUNK)], src_v)
        pltpu.sync_copy(edge_ref.at[1, pl.ds(off, CHUNK)], dst_v)

        def _scan(i, cv):
            dstv = dst_v[pl.ds(i * 16, 16)]
            srcv = src_v[pl.ds(i * 16, 16)]
            msk = (dstv >= lo) & (dstv < hi)
            cs = plsc.cumsum(jnp.where(msk, 1, 0))
            pos = (cs - 1 + cv) & (CAP - 1)
            plsc.store_scatter(csrc, [pos], srcv, mask=msk)
            plsc.store_scatter(cdst, [pos], dstv - lo, mask=msk)
            pc = plsc.all_reduce_population_count(msk)
            return cv + pc

        cntv = lax.fori_loop(0, CHUNK // 16, _scan, cntv)
        tot = cntv[0]
        gdone = gdonev[0]
        navail = tot // GB - gdone
        _run_groups(gdone, navail)
        return cntv, gdonev + navail

    cntv, gdonev = lax.fori_loop(
        0, E // CHUNK, _chunk,
        (jnp.zeros((16,), jnp.int32), jnp.zeros((16,), jnp.int32)))

    tot = cntv[0]
    gdone = gdonev[0]
    for k in range(GB // 16):
        ppos = (tot + k * 16 + lane) & (CAP - 1)
        plsc.store_scatter(csrc, [ppos], pad_src)
        plsc.store_scatter(cdst, [ppos], pad_dst)
    _run_groups(gdone, (tot + GB - 1) // GB - gdone)

    @pl.when(wid < NW - 1)
    def _():
        pltpu.sync_copy(acc.at[pl.ds(0, RPW)], out_ref.at[pl.ds(lo, RPW)])

    @pl.when(wid == NW - 1)
    def _():
        pltpu.sync_copy(acc.at[pl.ds(0, RPW_LAST)],
                        out_ref.at[pl.ds((NW - 1) * RPW, RPW_LAST)])


@functools.partial(
    pl.kernel,
    out_type=jax.ShapeDtypeStruct((N, D), jnp.float32),
    mesh=plsc.VectorSubcoreMesh(core_axis_name="c", subcore_axis_name="s",
                                num_cores=NC, num_subcores=NS),
    compiler_params=pltpu.CompilerParams(needs_layout_passes=False),
    scratch_types=[
        pltpu.VMEM((CHUNK,), jnp.int32),
        pltpu.VMEM((CHUNK,), jnp.int32),
        pltpu.VMEM((CAP,), jnp.int32),
        pltpu.VMEM((CAP,), jnp.int32),
        pltpu.VMEM((2, GB, D), jnp.float32),
        pltpu.VMEM((RPW_MAX + 1, D), jnp.float32),
        pltpu.SemaphoreType.DMA((2,)),
    ],
)
def _segmax(edge_ref, m_ref, out_ref, src_v, dst_v, csrc, cdst,
            rows_v, acc, sem):
    _segmax_body(edge_ref, m_ref, out_ref, src_v, dst_v, csrc, cdst,
                 rows_v, acc, sem)


def kernel(x, edge_index, W_emb, b_emb, W_pool1, b_pool1, W_self1, W_neigh1,
           bias1, W_pool2, b_pool2, W_self2, W_neigh2, bias2):
    h, m1 = _dense1(x, W_emb, b_emb, W_pool1, b_pool1)
    agg1 = _segmax(edge_index, m1)
    h2, m2 = _dense2(h, agg1, W_self1, W_neigh1, bias1, W_pool2, b_pool2)
    agg2 = _segmax(edge_index, m2)
    return _dense3(h2, agg2, W_self2, W_neigh2, bias2)


# speedup vs baseline: 1.2809x; 1.0734x over previous
import functools

import jax
import jax.numpy as jnp
from jax import lax
from jax.experimental import pallas as pl
from jax.experimental.pallas import tpu as pltpu
from jax.experimental.pallas import tpu_sc as plsc

N = 10000
E = 320000
D = 128

NC = 2
NS = 16
NW = NC * NS
RPW = 312
RPW_LAST = N - (NW - 1) * RPW
RPW_MAX = max(RPW, RPW_LAST)
CHUNK = 12800
GB = 64
CAP = 16384


_ROWS = 400


def _dense1_body(x_ref, wemb_ref, bemb_ref, wp_ref, bp_ref, h_ref, m_ref):
    h = jnp.dot(x_ref[:], wemb_ref[:], preferred_element_type=jnp.float32)
    h = h + bemb_ref[:]
    h_ref[:] = h
    m = jnp.dot(h, wp_ref[:], preferred_element_type=jnp.float32) + bp_ref[:]
    m_ref[:] = jnp.maximum(m, 0.0)


def _dense2_body(h_ref, agg_ref, ws_ref, wn_ref, b_ref, wp_ref, bp_ref,
                 h2_ref, m2_ref):
    h2 = (jnp.dot(h_ref[:], ws_ref[:], preferred_element_type=jnp.float32)
          + jnp.dot(agg_ref[:], wn_ref[:], preferred_element_type=jnp.float32)
          + b_ref[:])
    h2_ref[:] = h2
    m2 = jnp.dot(h2, wp_ref[:], preferred_element_type=jnp.float32) + bp_ref[:]
    m2_ref[:] = jnp.maximum(m2, 0.0)


def _dense3_body(h_ref, agg_ref, ws_ref, wn_ref, b_ref, out_ref):
    out_ref[:] = (jnp.dot(h_ref[:], ws_ref[:], preferred_element_type=jnp.float32)
                  + jnp.dot(agg_ref[:], wn_ref[:], preferred_element_type=jnp.float32)
                  + b_ref[:])


def _row_spec():
    return pl.BlockSpec((_ROWS, D), lambda i: (i, 0))


def _w_spec():
    return pl.BlockSpec((D, D), lambda i: (0, 0))


def _b_spec():
    return pl.BlockSpec((1, D), lambda i: (0, 0))


def _dense1(x, W_emb, b_emb, W_pool, b_pool):
    return pl.pallas_call(
        _dense1_body,
        grid=(N // _ROWS,),
        in_specs=[_row_spec(), _w_spec(), _b_spec(), _w_spec(), _b_spec()],
        out_specs=[_row_spec(), _row_spec()],
        out_shape=[jax.ShapeDtypeStruct((N, D), jnp.float32),
                   jax.ShapeDtypeStruct((N, D), jnp.float32)],
    )(x, W_emb, b_emb.reshape(1, D), W_pool, b_pool.reshape(1, D))


def _dense2(h, agg, W_self, W_neigh, bias, W_pool, b_pool):
    return pl.pallas_call(
        _dense2_body,
        grid=(N // _ROWS,),
        in_specs=[_row_spec(), _row_spec(), _w_spec(), _w_spec(), _b_spec(),
                  _w_spec(), _b_spec()],
        out_specs=[_row_spec(), _row_spec()],
        out_shape=[jax.ShapeDtypeStruct((N, D), jnp.float32),
                   jax.ShapeDtypeStruct((N, D), jnp.float32)],
    )(h, agg, W_self, W_neigh, bias.reshape(1, D), W_pool, b_pool.reshape(1, D))


def _dense3(h, agg, W_self, W_neigh, bias):
    return pl.pallas_call(
        _dense3_body,
        grid=(N // _ROWS,),
        in_specs=[_row_spec(), _row_spec(), _w_spec(), _w_spec(), _b_spec()],
        out_specs=_row_spec(),
        out_shape=jax.ShapeDtypeStruct((N, D), jnp.float32),
    )(h, agg, W_self, W_neigh, bias.reshape(1, D))


def _segmax_body(edge_ref, m_ref, out_ref, src_v, dst_v, csrc, cdst,
                 rows_v, acc, sem):
    cid = lax.axis_index("c")
    sid = lax.axis_index("s")
    wid = sid * NC + cid
    lo = wid * RPW
    hi = jnp.where(wid == NW - 1, N, lo + RPW)
    lane = jnp.arange(16, dtype=jnp.int32)
    zeros = jnp.zeros((16,), jnp.float32)
    pad_src = wid * 16 + lane

    def _zero(r, _):
        for cb in range(D // 16):
            acc[r, pl.ds(cb * 16, 16)] = zeros
        return 0
    lax.fori_loop(0, RPW_MAX + 1, _zero, 0)

    pad_dst = jnp.full((16,), RPW_MAX, jnp.int32)

    def _fire(g, slot):
        start = (g & (CAP // GB - 1)) * GB
        idx = csrc.at[pl.ds(start, GB)]
        return pltpu.make_async_copy(m_ref.at[idx], rows_v.at[slot],
                                     sem.at[slot])

    def _run_groups(g0, n):
        @pl.when(n > 0)
        def _():
            _fire(g0, 0).start()

        def _grp(i, _):
            slot = lax.rem(i, 2)

            @pl.when(i + 1 < n)
            def _():
                _fire(g0 + i + 1, 1 - slot).start()

            _fire(g0 + i, slot).wait()
            start = ((g0 + i) & (CAP // GB - 1)) * GB
            for sub in range(GB // 16):
                dstv = cdst[pl.ds(start + sub * 16, 16)]
                for e in range(16):
                    dstl = dstv[e]
                    r = sub * 16 + e
                    for cb in range(D // 16):
                        cur = acc[dstl, pl.ds(cb * 16, 16)]
                        val = rows_v[slot, r, pl.ds(cb * 16, 16)]
                        acc[dstl, pl.ds(cb * 16, 16)] = jnp.maximum(cur, val)
            return 0

        lax.fori_loop(0, n, _grp, 0)

    def _chunk(ch, carry):
        cntv, gdonev = carry
        off = ch * CHUNK
        pltpu.sync_copy(edge_ref.at[0, pl.ds(off, CHUNK)], src_v)
        pltpu.sync_copy(edge_ref.at[1, pl.ds(off, CHUNK)], dst_v)

        def _scan(i, cv):
            dstv = dst_v[pl.ds(i * 16, 16)]
            srcv = src_v[pl.ds(i * 16, 16)]
            msk = (dstv >= lo) & (dstv < hi)
            cs = plsc.cumsum(jnp.where(msk, 1, 0))
            pos = (cs - 1 + cv) & (CAP - 1)
            plsc.store_scatter(csrc, [pos], srcv, mask=msk)
            plsc.store_scatter(cdst, [pos], dstv - lo, mask=msk)
            pc = plsc.all_reduce_population_count(msk)
            return cv + pc

        cntv = lax.fori_loop(0, CHUNK // 16, _scan, cntv)
        tot = cntv[0]
        gdone = gdonev[0]
        navail = tot // GB - gdone
        _run_groups(gdone, navail)
        return cntv, gdonev + navail

    cntv, gdonev = lax.fori_loop(
        0, E // CHUNK, _chunk,
        (jnp.zeros((16,), jnp.int32), jnp.zeros((16,), jnp.int32)))

    tot = cntv[0]
    gdone = gdonev[0]
    for k in range(GB // 16):
        ppos = (tot + k * 16 + lane) & (CAP - 1)
        plsc.store_scatter(csrc, [ppos], pad_src)
        plsc.store_scatter(cdst, [ppos], pad_dst)
    _run_groups(gdone, (tot + GB - 1) // GB - gdone)

    @pl.when(wid < NW - 1)
    def _():
        pltpu.sync_copy(acc.at[pl.ds(0, RPW)], out_ref.at[pl.ds(lo, RPW)])

    @pl.when(wid == NW - 1)
    def _():
        pltpu.sync_copy(acc.at[pl.ds(0, RPW_LAST)],
                        out_ref.at[pl.ds((NW - 1) * RPW, RPW_LAST)])


@functools.partial(
    pl.kernel,
    out_type=jax.ShapeDtypeStruct((N, D), jnp.float32),
    mesh=plsc.VectorSubcoreMesh(core_axis_name="c", subcore_axis_name="s",
                                num_cores=NC, num_subcores=NS),
    compiler_params=pltpu.CompilerParams(needs_layout_passes=False),
    scratch_types=[
        pltpu.VMEM((CHUNK,), jnp.int32),
        pltpu.VMEM((CHUNK,), jnp.int32),
        pltpu.VMEM((CAP,), jnp.int32),
        pltpu.VMEM((CAP,), jnp.int32),
        pltpu.VMEM((2, GB, D), jnp.float32),
        pltpu.VMEM((RPW_MAX + 1, D), jnp.float32),
        pltpu.SemaphoreType.DMA((2,)),
    ],
)
def _segmax(edge_ref, m_ref, out_ref, src_v, dst_v, csrc, cdst,
            rows_v, acc, sem):
    _segmax_body(edge_ref, m_ref, out_ref, src_v, dst_v, csrc, cdst,
                 rows_v, acc, sem)


def kernel(x, edge_index, W_emb, b_emb, W_pool1, b_pool1, W_self1, W_neigh1,
           bias1, W_pool2, b_pool2, W_self2, W_neigh2, bias2):
    h, m1 = _dense1(x, W_emb, b_emb, W_pool1, b_pool1)
    agg1 = _segmax(edge_index, m1)
    h2, m2 = _dense2(h, agg1, W_self1, W_neigh1, bias1, W_pool2, b_pool2)
    agg2 = _segmax(edge_index, m2)
    return _dense3(h2, agg2, W_self2, W_neigh2, bias2)


# speedup vs baseline: 1.3018x; 1.0163x over previous
import functools

import jax
import jax.numpy as jnp
from jax import lax
from jax.experimental import pallas as pl
from jax.experimental.pallas import tpu as pltpu
from jax.experimental.pallas import tpu_sc as plsc

N = 10000
E = 320000
D = 128

NC = 2
NS = 16
NW = NC * NS
RPW = 312
RPW_LAST = N - (NW - 1) * RPW
RPW_MAX = max(RPW, RPW_LAST)
CHUNK = 16000
GB = 64
CAP = 16384


_ROWS = 400


def _dense1_body(x_ref, wemb_ref, bemb_ref, wp_ref, bp_ref, h_ref, m_ref):
    h = jnp.dot(x_ref[:], wemb_ref[:], preferred_element_type=jnp.float32)
    h = h + bemb_ref[:]
    h_ref[:] = h
    m = jnp.dot(h, wp_ref[:], preferred_element_type=jnp.float32) + bp_ref[:]
    m_ref[:] = jnp.maximum(m, 0.0)


def _dense2_body(h_ref, agg_ref, ws_ref, wn_ref, b_ref, wp_ref, bp_ref,
                 h2_ref, m2_ref):
    h2 = (jnp.dot(h_ref[:], ws_ref[:], preferred_element_type=jnp.float32)
          + jnp.dot(agg_ref[:], wn_ref[:], preferred_element_type=jnp.float32)
          + b_ref[:])
    h2_ref[:] = h2
    m2 = jnp.dot(h2, wp_ref[:], preferred_element_type=jnp.float32) + bp_ref[:]
    m2_ref[:] = jnp.maximum(m2, 0.0)


def _dense3_body(h_ref, agg_ref, ws_ref, wn_ref, b_ref, out_ref):
    out_ref[:] = (jnp.dot(h_ref[:], ws_ref[:], preferred_element_type=jnp.float32)
                  + jnp.dot(agg_ref[:], wn_ref[:], preferred_element_type=jnp.float32)
                  + b_ref[:])


def _row_spec():
    return pl.BlockSpec((_ROWS, D), lambda i: (i, 0))


def _w_spec():
    return pl.BlockSpec((D, D), lambda i: (0, 0))


def _b_spec():
    return pl.BlockSpec((1, D), lambda i: (0, 0))


def _dense1(x, W_emb, b_emb, W_pool, b_pool):
    return pl.pallas_call(
        _dense1_body,
        grid=(N // _ROWS,),
        in_specs=[_row_spec(), _w_spec(), _b_spec(), _w_spec(), _b_spec()],
        out_specs=[_row_spec(), _row_spec()],
        out_shape=[jax.ShapeDtypeStruct((N, D), jnp.float32),
                   jax.ShapeDtypeStruct((N, D), jnp.float32)],
    )(x, W_emb, b_emb.reshape(1, D), W_pool, b_pool.reshape(1, D))


def _dense2(h, agg, W_self, W_neigh, bias, W_pool, b_pool):
    return pl.pallas_call(
        _dense2_body,
        grid=(N // _ROWS,),
        in_specs=[_row_spec(), _row_spec(), _w_spec(), _w_spec(), _b_spec(),
                  _w_spec(), _b_spec()],
        out_specs=[_row_spec(), _row_spec()],
        out_shape=[jax.ShapeDtypeStruct((N, D), jnp.float32),
                   jax.ShapeDtypeStruct((N, D), jnp.float32)],
    )(h, agg, W_self, W_neigh, bias.reshape(1, D), W_pool, b_pool.reshape(1, D))


def _dense3(h, agg, W_self, W_neigh, bias):
    return pl.pallas_call(
        _dense3_body,
        grid=(N // _ROWS,),
        in_specs=[_row_spec(), _row_spec(), _w_spec(), _w_spec(), _b_spec()],
        out_specs=_row_spec(),
        out_shape=jax.ShapeDtypeStruct((N, D), jnp.float32),
    )(h, agg, W_self, W_neigh, bias.reshape(1, D))


def _segmax_body(edge_ref, m_ref, out_ref, src_v, dst_v, csrc, cdst,
                 rows_v, acc, sem):
    cid = lax.axis_index("c")
    sid = lax.axis_index("s")
    wid = sid * NC + cid
    lo = wid * RPW
    hi = jnp.where(wid == NW - 1, N, lo + RPW)
    lane = jnp.arange(16, dtype=jnp.int32)
    zeros = jnp.zeros((16,), jnp.float32)
    pad_src = wid * 16 + lane

    def _zero(r, _):
        for cb in range(D // 16):
            acc[r, pl.ds(cb * 16, 16)] = zeros
        return 0
    lax.fori_loop(0, RPW_MAX + 1, _zero, 0)

    pad_dst = jnp.full((16,), RPW_MAX, jnp.int32)

    def _fire(g, slot):
        start = (g & (CAP // GB - 1)) * GB
        idx = csrc.at[pl.ds(start, GB)]
        return pltpu.make_async_copy(m_ref.at[idx], rows_v.at[slot],
                                     sem.at[slot])

    def _run_groups(g0, n):
        @pl.when(n > 0)
        def _():
            _fire(g0, 0).start()

        def _grp(i, _):
            slot = lax.rem(i, 2)

            @pl.when(i + 1 < n)
            def _():
                _fire(g0 + i + 1, 1 - slot).start()

            _fire(g0 + i, slot).wait()
            start = ((g0 + i) & (CAP // GB - 1)) * GB
            for sub in range(GB // 16):
                dstv = cdst[pl.ds(start + sub * 16, 16)]
                for e in range(16):
                    dstl = dstv[e]
                    r = sub * 16 + e
                    for cb in range(D // 16):
                        cur = acc[dstl, pl.ds(cb * 16, 16)]
                        val = rows_v[slot, r, pl.ds(cb * 16, 16)]
                        acc[dstl, pl.ds(cb * 16, 16)] = jnp.maximum(cur, val)
            return 0

        lax.fori_loop(0, n, _grp, 0)

    def _chunk(ch, carry):
        cntv, gdonev = carry
        off = ch * CHUNK
        pltpu.sync_copy(edge_ref.at[0, pl.ds(off, CHUNK)], src_v)
        pltpu.sync_copy(edge_ref.at[1, pl.ds(off, CHUNK)], dst_v)

        def _scan(i, cv):
            dstv = dst_v[pl.ds(i * 16, 16)]
            srcv = src_v[pl.ds(i * 16, 16)]
            msk = (dstv >= lo) & (dstv < hi)
            cs = plsc.cumsum(jnp.where(msk, 1, 0))
            pos = (cs - 1 + cv) & (CAP - 1)
            plsc.store_scatter(csrc, [pos], srcv, mask=msk)
            plsc.store_scatter(cdst, [pos], dstv - lo, mask=msk)
            pc = plsc.all_reduce_population_count(msk)
            return cv + pc

        cntv = lax.fori_loop(0, CHUNK // 16, _scan, cntv)
        tot = cntv[0]
        gdone = gdonev[0]
        navail = tot // GB - gdone
        _run_groups(gdone, navail)
        return cntv, gdonev + navail

    cntv, gdonev = lax.fori_loop(
        0, E // CHUNK, _chunk,
        (jnp.zeros((16,), jnp.int32), jnp.zeros((16,), jnp.int32)))

    tot = cntv[0]
    gdone = gdonev[0]
    for k in range(GB // 16):
        ppos = (tot + k * 16 + lane) & (CAP - 1)
        plsc.store_scatter(csrc, [ppos], pad_src)
        plsc.store_scatter(cdst, [ppos], pad_dst)
    _run_groups(gdone, (tot + GB - 1) // GB - gdone)

    @pl.when(wid < NW - 1)
    def _():
        pltpu.sync_copy(acc.at[pl.ds(0, RPW)], out_ref.at[pl.ds(lo, RPW)])

    @pl.when(wid == NW - 1)
    def _():
        pltpu.sync_copy(acc.at[pl.ds(0, RPW_LAST)],
                        out_ref.at[pl.ds((NW - 1) * RPW, RPW_LAST)])


@functools.partial(
    pl.kernel,
    out_type=jax.ShapeDtypeStruct((N, D), jnp.float32),
    mesh=plsc.VectorSubcoreMesh(core_axis_name="c", subcore_axis_name="s",
                                num_cores=NC, num_subcores=NS),
    compiler_params=pltpu.CompilerParams(needs_layout_passes=False),
    scratch_types=[
        pltpu.VMEM((CHUNK,), jnp.int32),
        pltpu.VMEM((CHUNK,), jnp.int32),
        pltpu.VMEM((CAP,), jnp.int32),
        pltpu.VMEM((CAP,), jnp.int32),
        pltpu.VMEM((2, GB, D), jnp.float32),
        pltpu.VMEM((RPW_MAX + 1, D), jnp.float32),
        pltpu.SemaphoreType.DMA((2,)),
    ],
)
def _segmax(edge_ref, m_ref, out_ref, src_v, dst_v, csrc, cdst,
            rows_v, acc, sem):
    _segmax_body(edge_ref, m_ref, out_ref, src_v, dst_v, csrc, cdst,
                 rows_v, acc, sem)


def kernel(x, edge_index, W_emb, b_emb, W_pool1, b_pool1, W_self1, W_neigh1,
           bias1, W_pool2, b_pool2, W_self2, W_neigh2, bias2):
    h, m1 = _dense1(x, W_emb, b_emb, W_pool1, b_pool1)
    agg1 = _segmax(edge_index, m1)
    h2, m2 = _dense2(h, agg1, W_self1, W_neigh1, bias1, W_pool2, b_pool2)
    agg2 = _segmax(edge_index, m2)
    return _dense3(h2, agg2, W_self2, W_neigh2, bias2)
